# baseline (speedup 1.0000x reference)
import functools

import jax
import jax.numpy as jnp
from jax import lax
from jax.experimental import pallas as pl
from jax.experimental.pallas import tpu as pltpu
from jax.experimental.pallas import tpu_sc as plsc

VOCAB_P1 = 100001
EMB = 32
LEN_SEQ = 4
MAP_H = 16
MAP_W = 16
SEQ_SIZE = 20
BATCH = 32

PAIRS = BATCH * LEN_SEQ
SEGS_PER_PAIR = MAP_H * MAP_W
IDX_PER_PAIR = SEGS_PER_PAIR * SEQ_SIZE
NW = 32
PAIRS_PER_W = PAIRS // NW
IDX_ROW = 128
ROWS_PER_CHUNK = 640
SEGS_PER_CHUNK = ROWS_PER_CHUNK // SEQ_SIZE
CHUNKS = IDX_PER_PAIR // ROWS_PER_CHUNK
GATHERS_PER_CHUNK = ROWS_PER_CHUNK // IDX_ROW


def _tree_sum(vals):
    while len(vals) > 1:
        nxt = [vals[i] + vals[i + 1] for i in range(0, len(vals) - 1, 2)]
        if len(vals) % 2:
            nxt.append(vals[-1])
        vals = nxt
    return vals[0]


def _sc_body(x_hbm, emb_hbm, out_hbm, idx_v, rows_v, acc_v, sem):
    wid = lax.axis_index("s") * 2 + lax.axis_index("c")
    iota = lax.iota(jnp.int32, 16)
    sc0 = (iota * 2) * SEGS_PER_PAIR
    sc1 = sc0 + SEGS_PER_PAIR

    def start_chunk(c, buf):
        cps = []
        for j in range(GATHERS_PER_CHUNK):
            cp = pltpu.make_async_copy(
                emb_hbm.at[idx_v.at[c * GATHERS_PER_CHUNK + j]],
                rows_v.at[buf, pl.ds(j * IDX_ROW, IDX_ROW)],
                sem,
            )
            cp.start()
            cps.append(cp)
        return cps

    def pair_body(pi, _):
        p = wid * PAIRS_PER_W + pi
        pltpu.sync_copy(x_hbm.at[p], idx_v)
        pend = start_chunk(0, 0)
        for c in range(CHUNKS):
            cur = c % 2
            for cp in pend:
                cp.wait()
            if c + 1 < CHUNKS:
                pend = start_chunk(c + 1, 1 - cur)

            @plsc.parallel_loop(0, SEGS_PER_CHUNK, unroll=2)
            def seg_body(s, cur=cur, c=c):
                base = s * SEQ_SIZE
                rows = [rows_v[cur, base + k, :] for k in range(SEQ_SIZE)]
                l1 = [rows[2 * i] + rows[2 * i + 1] for i in range(10)]
                l2 = [l1[2 * i] + l1[2 * i + 1] for i in range(5)]
                parts = [
                    plsc.unpack(p, format=plsc.PackFormat.INTERLEAVED)
                    for p in l2
                ]
                a0 = _tree_sum([p[0] for p in parts])
                a1 = _tree_sum([p[1] for p in parts])
                seg = c * SEGS_PER_CHUNK + s
                plsc.store_scatter(acc_v, [sc0 + seg], a0)
                plsc.store_scatter(acc_v, [sc1 + seg], a1)

        pltpu.sync_copy(acc_v, out_hbm.at[p])
        return 0

    lax.fori_loop(0, PAIRS_PER_W, pair_body, 0)


@functools.partial(jax.jit, static_argnames=())
def kernel(x, embeddings):
    x3 = x.astype(jnp.int32).reshape(PAIRS, IDX_PER_PAIR // IDX_ROW, IDX_ROW)
    embeddings = embeddings.astype(jnp.bfloat16)
    mesh = plsc.VectorSubcoreMesh(core_axis_name="c", subcore_axis_name="s")
    out = pl.kernel(
        _sc_body,
        mesh=mesh,
        compiler_params=pltpu.CompilerParams(
            needs_layout_passes=False, use_tc_tiling_on_sc=False
        ),
        out_type=jax.ShapeDtypeStruct((PAIRS, EMB * SEGS_PER_PAIR), jnp.float32),
        scratch_types=[
            pltpu.VMEM((IDX_PER_PAIR // IDX_ROW, IDX_ROW), jnp.int32),
            pltpu.VMEM((2, ROWS_PER_CHUNK, EMB), jnp.bfloat16),
            pltpu.VMEM((EMB * SEGS_PER_PAIR,), jnp.float32),
            pltpu.SemaphoreType.DMA,
        ],
    )(x3, embeddings)
    return out.reshape(BATCH, LEN_SEQ * EMB, MAP_H, MAP_W)


# speedup vs baseline: 1.1195x; 1.1195x over previous
import functools

import jax
import jax.numpy as jnp
from jax import lax
from jax.experimental import pallas as pl
from jax.experimental.pallas import tpu as pltpu
from jax.experimental.pallas import tpu_sc as plsc

VOCAB_P1 = 100001
EMB = 32
LEN_SEQ = 4
MAP_H = 16
MAP_W = 16
SEQ_SIZE = 20
BATCH = 32

PAIRS = BATCH * LEN_SEQ
SEGS_PER_PAIR = MAP_H * MAP_W
IDX_PER_PAIR = SEGS_PER_PAIR * SEQ_SIZE
NW = 32
PAIRS_PER_W = PAIRS // NW
IDX_ROW = 128
ROWS_PER_CHUNK = 640
SEGS_PER_CHUNK = ROWS_PER_CHUNK // SEQ_SIZE
CHUNKS = IDX_PER_PAIR // ROWS_PER_CHUNK
GATHERS_PER_CHUNK = ROWS_PER_CHUNK // IDX_ROW


def _tree_sum(vals):
    while len(vals) > 1:
        nxt = [vals[i] + vals[i + 1] for i in range(0, len(vals) - 1, 2)]
        if len(vals) % 2:
            nxt.append(vals[-1])
        vals = nxt
    return vals[0]


def _sc_body(x_hbm, emb_hbm, out_hbm, idx_v, rows_v, acc_v, sem):
    wid = lax.axis_index("s") * 2 + lax.axis_index("c")
    iota = lax.iota(jnp.int32, 16)
    sc0 = iota * SEGS_PER_PAIR
    sc1 = sc0 + 16 * SEGS_PER_PAIR
    def start_chunk(c, buf):
        cps = []
        for j in range(GATHERS_PER_CHUNK):
            cp = pltpu.make_async_copy(
                emb_hbm.at[idx_v.at[pl.ds((c * GATHERS_PER_CHUNK + j) * IDX_ROW, IDX_ROW)]],
                rows_v.at[buf, pl.ds(j * IDX_ROW, IDX_ROW)],
                sem,
            )
            cp.start()
            cps.append(cp)
        return cps

    def pair_body(pi, _):
        p = wid * PAIRS_PER_W + pi
        pltpu.sync_copy(x_hbm.at[p], idx_v)
        pend = start_chunk(0, 0)
        for c in range(CHUNKS):
            cur = c % 2
            for cp in pend:
                cp.wait()
            if c + 1 < CHUNKS:
                pend = start_chunk(c + 1, 1 - cur)

            @plsc.parallel_loop(0, SEGS_PER_CHUNK, unroll=2)
            def seg_body(s, cur=cur, c=c):
                base = s * SEQ_SIZE
                a0 = _tree_sum(
                    [rows_v[cur, base + k, pl.ds(0, 16)] for k in range(SEQ_SIZE)]
                )
                a1 = _tree_sum(
                    [rows_v[cur, base + k, pl.ds(16, 16)] for k in range(SEQ_SIZE)]
                )
                seg = c * SEGS_PER_CHUNK + s
                plsc.store_scatter(acc_v, [sc0 + seg], a0)
                plsc.store_scatter(acc_v, [sc1 + seg], a1)

        pltpu.sync_copy(acc_v, out_hbm.at[p])
        return 0

    lax.fori_loop(0, PAIRS_PER_W, pair_body, 0)


@functools.partial(jax.jit, static_argnames=())
def kernel(x, embeddings):
    x = x.astype(jnp.int32).reshape(PAIRS, IDX_PER_PAIR)
    mesh = plsc.VectorSubcoreMesh(core_axis_name="c", subcore_axis_name="s")
    out = pl.kernel(
        _sc_body,
        mesh=mesh,
        compiler_params=pltpu.CompilerParams(
            needs_layout_passes=False, use_tc_tiling_on_sc=False
        ),
        out_type=jax.ShapeDtypeStruct((PAIRS, EMB * SEGS_PER_PAIR), jnp.float32),
        scratch_types=[
            pltpu.VMEM((IDX_PER_PAIR,), jnp.int32),
            pltpu.VMEM((2, ROWS_PER_CHUNK, EMB), jnp.float32),
            pltpu.VMEM((EMB * SEGS_PER_PAIR,), jnp.float32),
            pltpu.SemaphoreType.DMA,
        ],
    )(x, embeddings)
    return out.reshape(BATCH, LEN_SEQ * EMB, MAP_H, MAP_W)


# speedup vs baseline: 1.4717x; 1.3145x over previous
import functools

import jax
import jax.numpy as jnp
from jax import lax
from jax.experimental import pallas as pl
from jax.experimental.pallas import tpu as pltpu
from jax.experimental.pallas import tpu_sc as plsc

VOCAB_P1 = 100001
EMB = 32
LEN_SEQ = 4
MAP_H = 16
MAP_W = 16
SEQ_SIZE = 20
BATCH = 32

PAIRS = BATCH * LEN_SEQ
SEGS_PER_PAIR = MAP_H * MAP_W
IDX_PER_PAIR = SEGS_PER_PAIR * SEQ_SIZE
NW = 32
PAIRS_PER_W = PAIRS // NW
IDX_ROW = 128
ROWS_PER_CHUNK = 640
SEGS_PER_CHUNK = ROWS_PER_CHUNK // SEQ_SIZE
CHUNKS = IDX_PER_PAIR // ROWS_PER_CHUNK
GATHERS_PER_CHUNK = ROWS_PER_CHUNK // IDX_ROW


def _tree_sum(vals):
    while len(vals) > 1:
        nxt = [vals[i] + vals[i + 1] for i in range(0, len(vals) - 1, 2)]
        if len(vals) % 2:
            nxt.append(vals[-1])
        vals = nxt
    return vals[0]


def _sc_body(x_hbm, emb_hbm, out_hbm, idx_v, rows_v, acc_v, sem):
    wid = lax.axis_index("s") * 2 + lax.axis_index("c")
    def start_chunk(c, buf):
        cps = []
        for j in range(GATHERS_PER_CHUNK):
            cp = pltpu.make_async_copy(
                emb_hbm.at[idx_v.at[pl.ds((c * GATHERS_PER_CHUNK + j) * IDX_ROW, IDX_ROW)]],
                rows_v.at[buf, pl.ds(j * IDX_ROW, IDX_ROW)],
                sem,
            )
            cp.start()
            cps.append(cp)
        return cps

    def pair_body(pi, _):
        p = wid * PAIRS_PER_W + pi
        pltpu.sync_copy(x_hbm.at[p], idx_v)
        pend = start_chunk(0, 0)
        for c in range(CHUNKS):
            cur = c % 2
            for cp in pend:
                cp.wait()
            if c + 1 < CHUNKS:
                pend = start_chunk(c + 1, 1 - cur)

            @plsc.parallel_loop(0, SEGS_PER_CHUNK, unroll=2)
            def seg_body(s, cur=cur, c=c):
                base = s * SEQ_SIZE
                a0 = _tree_sum(
                    [rows_v[cur, base + k, pl.ds(0, 16)] for k in range(SEQ_SIZE)]
                )
                a1 = _tree_sum(
                    [rows_v[cur, base + k, pl.ds(16, 16)] for k in range(SEQ_SIZE)]
                )
                seg = c * SEGS_PER_CHUNK + s
                acc_v[seg, pl.ds(0, 16)] = a0
                acc_v[seg, pl.ds(16, 16)] = a1

        b = p >> 2
        l = p & 3
        pltpu.sync_copy(
            acc_v, out_hbm.at[b, :, pl.ds(l * EMB, EMB)]
        )
        return 0

    lax.fori_loop(0, PAIRS_PER_W, pair_body, 0)


@functools.partial(jax.jit, static_argnames=())
def kernel(x, embeddings):
    x = x.astype(jnp.int32).reshape(PAIRS, IDX_PER_PAIR)
    mesh = plsc.VectorSubcoreMesh(core_axis_name="c", subcore_axis_name="s")
    out = pl.kernel(
        _sc_body,
        mesh=mesh,
        compiler_params=pltpu.CompilerParams(
            needs_layout_passes=False, use_tc_tiling_on_sc=False
        ),
        out_type=jax.ShapeDtypeStruct(
            (BATCH, SEGS_PER_PAIR, LEN_SEQ * EMB), jnp.float32
        ),
        scratch_types=[
            pltpu.VMEM((IDX_PER_PAIR,), jnp.int32),
            pltpu.VMEM((2, ROWS_PER_CHUNK, EMB), jnp.float32),
            pltpu.VMEM((SEGS_PER_PAIR, EMB), jnp.float32),
            pltpu.SemaphoreType.DMA,
        ],
    )(x, embeddings)
    return out.reshape(BATCH, MAP_H, MAP_W, LEN_SEQ * EMB).transpose(0, 3, 1, 2)
